# baseline (speedup 1.0000x reference)
import functools

import jax
import jax.numpy as jnp
from jax import lax
from jax.experimental import pallas as pl
from jax.experimental.pallas import tpu as pltpu
from jax.experimental.pallas import tpu_sc as plsc

N = 10000
E = 160000
D = 128
DW = D // 2
PD = 16
F = D + 2 * PD
CNT_LANE = 8
CH = 128
NCH = E // CH
NC = 2
NS = 16
NW = NC * NS
STRIPE = N // NS
DT = 0.001
DH = 0.001

BE = 2000
BN = 2000

_mesh = plsc.VectorSubcoreMesh(core_axis_name="c", subcore_axis_name="s")


@functools.partial(
    pl.kernel,
    out_type=(
        jax.ShapeDtypeStruct((E, DW), jnp.int32),
        jax.ShapeDtypeStruct((E, DW), jnp.int32),
        jax.ShapeDtypeStruct((E, PD), jnp.float32),
    ),
    mesh=_mesh,
    scratch_types=[
        pltpu.VMEM((CH,), jnp.int32),
        pltpu.VMEM((CH,), jnp.int32),
        pltpu.VMEM((CH, DW), jnp.int32),
        pltpu.VMEM((CH, DW), jnp.int32),
        pltpu.VMEM((CH, PD), jnp.float32),
        pltpu.VMEM((CH, PD), jnp.float32),
        pltpu.VMEM((CH, PD), jnp.float32),
        pltpu.SemaphoreType.DMA,
        pltpu.SemaphoreType.DMA,
        pltpu.SemaphoreType.DMA,
        pltpu.SemaphoreType.DMA,
    ],
    compiler_params=pltpu.CompilerParams(use_tc_tiling_on_sc=False),
)
def _sc_gather(h_hbm, posp_hbm, row_hbm, col_hbm, hr_hbm, hc_hbm, pd_hbm,
               idxr, idxc, hrv, hcv, prv, pcv, pdv, s0, s1, s2, s3):
    wid = lax.axis_index("s") * NC + lax.axis_index("c")
    nck = (NCH - wid + NW - 1) // NW

    def chunk(k, _):
        j = wid + k * NW
        base = j * CH
        pltpu.sync_copy(row_hbm.at[j], idxr)
        pltpu.sync_copy(col_hbm.at[j], idxc)
        cp0 = pltpu.async_copy(h_hbm.at[idxr], hrv, s0)
        cp1 = pltpu.async_copy(h_hbm.at[idxc], hcv, s1)
        cp2 = pltpu.async_copy(posp_hbm.at[idxr], prv, s2)
        cp3 = pltpu.async_copy(posp_hbm.at[idxc], pcv, s3)
        cp2.wait()
        cp3.wait()

        def drow(i, carry):
            pdv[i, :] = prv[i, :] - pcv[i, :]
            return carry

        lax.fori_loop(0, CH, drow, 0, unroll=4)
        cp0.wait()
        cp1.wait()
        pltpu.sync_copy(hrv, hr_hbm.at[pl.ds(base, CH)])
        pltpu.sync_copy(hcv, hc_hbm.at[pl.ds(base, CH)])
        pltpu.sync_copy(pdv, pd_hbm.at[pl.ds(base, CH)])
        return _

    lax.fori_loop(0, nck, chunk, 0)


@functools.partial(
    pl.kernel,
    out_type=jax.ShapeDtypeStruct((NC, N, F), jnp.bfloat16),
    mesh=_mesh,
    scratch_types=[
        pltpu.VMEM((CH, F), jnp.bfloat16),
        pltpu.VMEM((CH,), jnp.int32),
        pltpu.VMEM_SHARED((N, F), jnp.bfloat16),
    ],
    compiler_params=pltpu.CompilerParams(use_tc_tiling_on_sc=False),
)
def _sc_scatter(ef_hbm, row_hbm, z_hbm, parts_hbm, efv, idxv, acc):
    cid = lax.axis_index("c")
    sid = lax.axis_index("s")
    wid = sid * NC + cid
    pltpu.sync_copy(z_hbm, acc.at[pl.ds(sid * STRIPE, STRIPE)])
    plsc.subcore_barrier()

    nck = (NCH - wid + NW - 1) // NW

    def chunk(k, carry):
        j = wid + k * NW
        pltpu.sync_copy(row_hbm.at[j], idxv)
        pltpu.sync_copy(ef_hbm.at[pl.ds(j * CH, CH)], efv)
        pltpu.sync_copy(efv, acc.at[idxv], add=True)
        return carry

    lax.fori_loop(0, nck, chunk, 0)
    plsc.subcore_barrier()
    pltpu.sync_copy(acc.at[pl.ds(sid * STRIPE, STRIPE)],
                    parts_hbm.at[cid, pl.ds(sid * STRIPE, STRIPE)])


def _edge_body(hr, hc, pd, W1a, W1b, w1r, b1, W2, b2, Wc1, bc1, Wc2, out):
    f32 = jnp.float32
    pdv = pd[...]
    radial = jnp.sum(pdv * pdv, axis=1, keepdims=True)
    x = (jnp.dot(hr[...], W1a[...], preferred_element_type=f32)
         + jnp.dot(hc[...], W1b[...], preferred_element_type=f32)
         + radial * w1r[...] + b1[...])
    x = x * jax.nn.sigmoid(x)
    x = jnp.dot(x.astype(jnp.bfloat16), W2[...], preferred_element_type=f32)
    x = x + b2[...]
    e = x * jax.nn.sigmoid(x)
    eb = e.astype(jnp.bfloat16)
    y = jnp.dot(eb, Wc1[...], preferred_element_type=f32) + bc1[...]
    y = y * jax.nn.sigmoid(y)
    cw = y @ Wc2[...]
    tr = jnp.clip(pdv * cw, -100.0, 100.0)
    lane = lax.broadcasted_iota(jnp.int32, (1, PD), 1)
    tr = jnp.where(lane == CNT_LANE, 1.0, tr)
    trb = jnp.concatenate([tr, jnp.zeros_like(tr)], axis=1).astype(jnp.bfloat16)
    out[...] = jnp.concatenate([eb, trb], axis=1)


def _tc_edge(hr, hc, pd, W1a, W1b, w1r, b1, W2, b2, Wc1, bc1, Wc2):
    nb = E // BE
    wspec = lambda shape: pl.BlockSpec(shape, lambda i: (0, 0))
    return pl.pallas_call(
        _edge_body,
        grid=(nb,),
        in_specs=[
            pl.BlockSpec((BE, D), lambda i: (i, 0)),
            pl.BlockSpec((BE, D), lambda i: (i, 0)),
            pl.BlockSpec((BE, PD), lambda i: (i, 0)),
            wspec((D, D)), wspec((D, D)), wspec((1, D)), wspec((1, D)),
            wspec((D, D)), wspec((1, D)),
            wspec((D, D)), wspec((1, D)), wspec((D, 1)),
        ],
        out_specs=pl.BlockSpec((BE, F), lambda i: (i, 0)),
        out_shape=jax.ShapeDtypeStruct((E, F), jnp.bfloat16),
        compiler_params=pltpu.CompilerParams(
            dimension_semantics=("arbitrary",)),
    )(hr, hc, pd, W1a, W1b, w1r, b1, W2, b2, Wc1, bc1, Wc2)


def _node_body(h, g, velp, posp, parts, Wv1, bv1, Wv2, bv2,
               Wn1a, Wn1b, bn1, Wn2, bn2,
               h2, g2, velp2, posp2, ldj):
    ps = (parts[0].astype(jnp.float32)
          + parts[1].astype(jnp.float32))
    agg = ps[:, :D]
    st = ps[:, D:D + PD]
    lane = lax.broadcasted_iota(jnp.int32, (1, PD), 1)
    cnt = jnp.sum(jnp.where(lane == CNT_LANE, st, 0.0), axis=1, keepdims=True)
    force = st / jnp.clip(cnt, 1.0, None)
    force = jnp.where(lane < 3, force, 0.0)

    hv = h[...]
    sv = hv @ Wv1[...] + bv1[...]
    sv = sv * jax.nn.sigmoid(sv)
    sv = sv @ Wv2[...] + bv2[...]

    x = hv @ Wn1a[...] + agg @ Wn1b[...] + bn1[...]
    x = x * jax.nn.sigmoid(x)
    no = x @ Wn2[...] + bn2[...]

    vel_new = jnp.exp(sv) * velp[...] + force * DT
    posp2[...] = posp[...] + vel_new * DT
    velp2[...] = vel_new
    g_new = g[...] + no * DH
    g2[...] = g_new
    h2[...] = hv + g_new * DH

    @pl.when(pl.program_id(0) == 0)
    def _():
        ldj[...] = jnp.zeros_like(ldj)

    ldj[...] += jnp.sum(sv)


def _tc_node(h, g, velp, posp, parts, Wv1, bv1, Wv2, bv2,
             Wn1a, Wn1b, bn1, Wn2, bn2):
    nb = N // BN
    wspec = lambda shape: pl.BlockSpec(shape, lambda i: tuple(0 for _ in shape))
    return pl.pallas_call(
        _node_body,
        grid=(nb,),
        in_specs=[
            pl.BlockSpec((BN, D), lambda i: (i, 0)),
            pl.BlockSpec((BN, D), lambda i: (i, 0)),
            pl.BlockSpec((BN, PD), lambda i: (i, 0)),
            pl.BlockSpec((BN, PD), lambda i: (i, 0)),
            pl.BlockSpec((NC, BN, F), lambda i: (0, i, 0)),
            wspec((D, D)), wspec((1, D)), wspec((D, 1)), wspec((1, 1)),
            wspec((D, D)), wspec((D, D)), wspec((1, D)),
            wspec((D, D)), wspec((1, D)),
        ],
        out_specs=[
            pl.BlockSpec((BN, D), lambda i: (i, 0)),
            pl.BlockSpec((BN, D), lambda i: (i, 0)),
            pl.BlockSpec((BN, PD), lambda i: (i, 0)),
            pl.BlockSpec((BN, PD), lambda i: (i, 0)),
            pl.BlockSpec((1, 1), lambda i: (0, 0)),
        ],
        out_shape=[
            jax.ShapeDtypeStruct((N, D), jnp.float32),
            jax.ShapeDtypeStruct((N, D), jnp.float32),
            jax.ShapeDtypeStruct((N, PD), jnp.float32),
            jax.ShapeDtypeStruct((N, PD), jnp.float32),
            jax.ShapeDtypeStruct((1, 1), jnp.float32),
        ],
        compiler_params=pltpu.CompilerParams(
            dimension_semantics=("arbitrary",)),
    )(h, g, velp, posp, parts, Wv1, bv1, Wv2, bv2,
      Wn1a, Wn1b, bn1, Wn2, bn2)


def _as_i32(x):
    return lax.bitcast_convert_type(
        x.reshape(x.shape[0], -1, 2), jnp.int32)


def _as_bf16(x):
    return lax.bitcast_convert_type(x, jnp.bfloat16).reshape(x.shape[0], -1)


def kernel(h, pos, g, vel, edge_index, W_e1, b_e1, W_e2, b_e2, W_n1, b_n1,
           W_n2, b_n2, W_c1, b_c1, W_c2, W_v1, b_v1, W_v2, b_v2):
    bf = jnp.bfloat16
    row2 = edge_index[0].reshape(NCH, CH)
    col2 = edge_index[1].reshape(NCH, CH)
    zpad = jnp.zeros((N, PD - 3), jnp.float32)
    posp = jnp.concatenate([pos, zpad], axis=1)
    velp = jnp.concatenate([vel, zpad], axis=1)
    zz = jnp.zeros((STRIPE, F), bf)

    ldj = jnp.zeros((), jnp.float32)
    for i in range(2):
        W1a, W1b, w1r = W_e1[i, :D].astype(bf), W_e1[i, D:2 * D].astype(bf), W_e1[i, 2 * D:]
        b1 = b_e1[i].reshape(1, D)
        W2, b2 = W_e2[i].astype(bf), b_e2[i].reshape(1, D)
        Wc1, bc1, Wc2 = W_c1[i].astype(bf), b_c1[i].reshape(1, D), W_c2[i]
        Wv1, bv1 = W_v1[i], b_v1[i].reshape(1, D)
        Wv2, bv2 = W_v2[i], b_v2[i].reshape(1, 1)
        Wn1a, Wn1b = W_n1[i, :D], W_n1[i, D:]
        bn1, Wn2, bn2 = b_n1[i].reshape(1, D), W_n2[i], b_n2[i].reshape(1, D)

        htbl = _as_i32(h.astype(bf))
        hr, hc, pd = _sc_gather(htbl, posp, row2, col2)
        ef = _tc_edge(_as_bf16(hr), _as_bf16(hc), pd,
                      W1a, W1b, w1r, b1, W2, b2, Wc1, bc1, Wc2)
        parts = _sc_scatter(ef, row2, zz)
        h, g, velp, posp, lds = _tc_node(h, g, velp, posp, parts,
                                         Wv1, bv1, Wv2, bv2,
                                         Wn1a, Wn1b, bn1, Wn2, bn2)
        ldj = ldj + lds[0, 0]

    return (h, g, posp[:, :3], velp[:, :3], ldj)


# speedup vs baseline: 1.7490x; 1.7490x over previous
import functools

import jax
import jax.numpy as jnp
from jax import lax
from jax.experimental import pallas as pl
from jax.experimental.pallas import tpu as pltpu
from jax.experimental.pallas import tpu_sc as plsc

N = 10000
E = 160000
D = 128
PD = 16
F = D + PD
CNT_LANE = 8
CH = 128
NCH = E // CH
NC = 2
NS = 16
NW = NC * NS
STRIPE = N // NS
DT = 0.001
DH = 0.001

BE = 2000
BN = 2000

_mesh = plsc.VectorSubcoreMesh(core_axis_name="c", subcore_axis_name="s")


@functools.partial(
    pl.kernel,
    out_type=(
        jax.ShapeDtypeStruct((E, D), jnp.float32),
        jax.ShapeDtypeStruct((E, PD), jnp.float32),
    ),
    mesh=_mesh,
    scratch_types=[
        pltpu.VMEM((CH,), jnp.int32),
        pltpu.VMEM((CH,), jnp.int32),
        pltpu.VMEM((CH, D), jnp.float32),
        pltpu.VMEM((CH, D), jnp.float32),
        pltpu.VMEM((CH, PD), jnp.float32),
        pltpu.VMEM((CH, PD), jnp.float32),
        pltpu.SemaphoreType.DMA,
        pltpu.SemaphoreType.DMA,
        pltpu.SemaphoreType.DMA,
        pltpu.SemaphoreType.DMA,
    ],
    compiler_params=pltpu.CompilerParams(use_tc_tiling_on_sc=False),
)
def _sc_gather(a_hbm, b_hbm, posp_hbm, row_hbm, col_hbm, s_hbm, pd_hbm,
               idxr, idxc, av, bv, prv, pcv, s0, s1, s2, s3):
    wid = lax.axis_index("s") * NC + lax.axis_index("c")
    nck = (NCH - wid + NW - 1) // NW

    def chunk(k, carry):
        j = wid + k * NW
        base = j * CH
        pltpu.sync_copy(row_hbm.at[j], idxr)
        pltpu.sync_copy(col_hbm.at[j], idxc)
        cp0 = pltpu.async_copy(a_hbm.at[idxr], av, s0)
        cp1 = pltpu.async_copy(b_hbm.at[idxc], bv, s1)
        cp2 = pltpu.async_copy(posp_hbm.at[idxr], prv, s2)
        cp3 = pltpu.async_copy(posp_hbm.at[idxc], pcv, s3)
        cp2.wait()
        cp3.wait()

        def drow(i, c2):
            prv[i, :] = prv[i, :] - pcv[i, :]
            return c2

        lax.fori_loop(0, CH, drow, 0, unroll=4)
        cp0.wait()
        cp1.wait()

        def srow(i, c2):
            for q in range(8):
                av[i, pl.ds(q * 16, 16)] = (av[i, pl.ds(q * 16, 16)]
                                            + bv[i, pl.ds(q * 16, 16)])
            return c2

        lax.fori_loop(0, CH, srow, 0, unroll=2)
        pltpu.sync_copy(av, s_hbm.at[pl.ds(base, CH)])
        pltpu.sync_copy(prv, pd_hbm.at[pl.ds(base, CH)])
        return carry

    lax.fori_loop(0, nck, chunk, 0)


@functools.partial(
    pl.kernel,
    out_type=jax.ShapeDtypeStruct((NC, N, F), jnp.float32),
    mesh=_mesh,
    scratch_types=[
        pltpu.VMEM((CH, F), jnp.float32),
        pltpu.VMEM((CH,), jnp.int32),
        pltpu.VMEM_SHARED((N, F), jnp.float32),
    ],
    compiler_params=pltpu.CompilerParams(use_tc_tiling_on_sc=False),
)
def _sc_scatter(ef_hbm, row_hbm, z_hbm, parts_hbm, efv, idxv, acc):
    cid = lax.axis_index("c")
    sid = lax.axis_index("s")
    wid = sid * NC + cid
    pltpu.sync_copy(z_hbm, acc.at[pl.ds(sid * STRIPE, STRIPE)])
    plsc.subcore_barrier()

    nck = (NCH - wid + NW - 1) // NW

    def chunk(k, carry):
        j = wid + k * NW
        pltpu.sync_copy(row_hbm.at[j], idxv)
        pltpu.sync_copy(ef_hbm.at[pl.ds(j * CH, CH)], efv)
        pltpu.sync_copy(efv, acc.at[idxv], add=True)
        return carry

    lax.fori_loop(0, nck, chunk, 0)
    plsc.subcore_barrier()
    pltpu.sync_copy(acc.at[pl.ds(sid * STRIPE, STRIPE)],
                    parts_hbm.at[cid, pl.ds(sid * STRIPE, STRIPE)])


def _prep_body(h, W1a, W1b, b1, a_out, b_out):
    hb = h[...].astype(jnp.bfloat16)
    a_out[...] = (jnp.dot(hb, W1a[...], preferred_element_type=jnp.float32)
                  + b1[...])
    b_out[...] = jnp.dot(hb, W1b[...], preferred_element_type=jnp.float32)


def _tc_prep(h, W1a, W1b, b1):
    nb = N // BN
    wspec = lambda shape: pl.BlockSpec(shape, lambda i: (0, 0))
    return pl.pallas_call(
        _prep_body,
        grid=(nb,),
        in_specs=[
            pl.BlockSpec((BN, D), lambda i: (i, 0)),
            wspec((D, D)), wspec((D, D)), wspec((1, D)),
        ],
        out_specs=[
            pl.BlockSpec((BN, D), lambda i: (i, 0)),
            pl.BlockSpec((BN, D), lambda i: (i, 0)),
        ],
        out_shape=[
            jax.ShapeDtypeStruct((N, D), jnp.float32),
            jax.ShapeDtypeStruct((N, D), jnp.float32),
        ],
        compiler_params=pltpu.CompilerParams(
            dimension_semantics=("arbitrary",)),
    )(h, W1a, W1b, b1)


def _edge_body(sv, pd, w1r, W2, b2, Wc1, bc1, Wc2, out):
    f32 = jnp.float32
    bf = jnp.bfloat16
    pdv = pd[...]
    radial = jnp.sum(pdv * pdv, axis=1, keepdims=True)
    x = sv[...] + radial * w1r[...]
    x = x * jax.nn.sigmoid(x)
    x = jnp.dot(x.astype(bf), W2[...], preferred_element_type=f32) + b2[...]
    e = x * jax.nn.sigmoid(x)
    eb = e.astype(bf)
    y = jnp.dot(eb, Wc1[...], preferred_element_type=f32) + bc1[...]
    y = y * jax.nn.sigmoid(y)
    cw = y @ Wc2[...]
    tr = jnp.clip(pdv * cw, -100.0, 100.0)
    lane = lax.broadcasted_iota(jnp.int32, (1, PD), 1)
    tr = jnp.where(lane == CNT_LANE, 1.0, tr)
    out[...] = jnp.concatenate([e, tr], axis=1)


def _tc_edge(sv, pd, w1r, W2, b2, Wc1, bc1, Wc2):
    nb = E // BE
    wspec = lambda shape: pl.BlockSpec(shape, lambda i: (0, 0))
    return pl.pallas_call(
        _edge_body,
        grid=(nb,),
        in_specs=[
            pl.BlockSpec((BE, D), lambda i: (i, 0)),
            pl.BlockSpec((BE, PD), lambda i: (i, 0)),
            wspec((1, D)),
            wspec((D, D)), wspec((1, D)),
            wspec((D, D)), wspec((1, D)), wspec((D, 1)),
        ],
        out_specs=pl.BlockSpec((BE, F), lambda i: (i, 0)),
        out_shape=jax.ShapeDtypeStruct((E, F), jnp.float32),
        compiler_params=pltpu.CompilerParams(
            dimension_semantics=("arbitrary",)),
    )(sv, pd, w1r, W2, b2, Wc1, bc1, Wc2)


def _node_body(h, g, velp, posp, parts, Wv1, bv1, Wv2, bv2,
               Wn1a, Wn1b, bn1, Wn2, bn2,
               h2, g2, velp2, posp2, ldj):
    ps = parts[0] + parts[1]
    agg = ps[:, :D]
    st = ps[:, D:]
    lane = lax.broadcasted_iota(jnp.int32, (1, PD), 1)
    cnt = jnp.sum(jnp.where(lane == CNT_LANE, st, 0.0), axis=1, keepdims=True)
    force = st / jnp.clip(cnt, 1.0, None)
    force = jnp.where(lane < 3, force, 0.0)

    hv = h[...]
    sv = hv @ Wv1[...] + bv1[...]
    sv = sv * jax.nn.sigmoid(sv)
    sv = sv @ Wv2[...] + bv2[...]

    x = hv @ Wn1a[...] + agg @ Wn1b[...] + bn1[...]
    x = x * jax.nn.sigmoid(x)
    no = x @ Wn2[...] + bn2[...]

    vel_new = jnp.exp(sv) * velp[...] + force * DT
    posp2[...] = posp[...] + vel_new * DT
    velp2[...] = vel_new
    g_new = g[...] + no * DH
    g2[...] = g_new
    h2[...] = hv + g_new * DH

    @pl.when(pl.program_id(0) == 0)
    def _():
        ldj[...] = jnp.zeros_like(ldj)

    ldj[...] += jnp.sum(sv)


def _tc_node(h, g, velp, posp, parts, Wv1, bv1, Wv2, bv2,
             Wn1a, Wn1b, bn1, Wn2, bn2):
    nb = N // BN
    wspec = lambda shape: pl.BlockSpec(shape, lambda i: tuple(0 for _ in shape))
    return pl.pallas_call(
        _node_body,
        grid=(nb,),
        in_specs=[
            pl.BlockSpec((BN, D), lambda i: (i, 0)),
            pl.BlockSpec((BN, D), lambda i: (i, 0)),
            pl.BlockSpec((BN, PD), lambda i: (i, 0)),
            pl.BlockSpec((BN, PD), lambda i: (i, 0)),
            pl.BlockSpec((NC, BN, F), lambda i: (0, i, 0)),
            wspec((D, D)), wspec((1, D)), wspec((D, 1)), wspec((1, 1)),
            wspec((D, D)), wspec((D, D)), wspec((1, D)),
            wspec((D, D)), wspec((1, D)),
        ],
        out_specs=[
            pl.BlockSpec((BN, D), lambda i: (i, 0)),
            pl.BlockSpec((BN, D), lambda i: (i, 0)),
            pl.BlockSpec((BN, PD), lambda i: (i, 0)),
            pl.BlockSpec((BN, PD), lambda i: (i, 0)),
            pl.BlockSpec((1, 1), lambda i: (0, 0)),
        ],
        out_shape=[
            jax.ShapeDtypeStruct((N, D), jnp.float32),
            jax.ShapeDtypeStruct((N, D), jnp.float32),
            jax.ShapeDtypeStruct((N, PD), jnp.float32),
            jax.ShapeDtypeStruct((N, PD), jnp.float32),
            jax.ShapeDtypeStruct((1, 1), jnp.float32),
        ],
        compiler_params=pltpu.CompilerParams(
            dimension_semantics=("arbitrary",)),
    )(h, g, velp, posp, parts, Wv1, bv1, Wv2, bv2,
      Wn1a, Wn1b, bn1, Wn2, bn2)


def kernel(h, pos, g, vel, edge_index, W_e1, b_e1, W_e2, b_e2, W_n1, b_n1,
           W_n2, b_n2, W_c1, b_c1, W_c2, W_v1, b_v1, W_v2, b_v2):
    bf = jnp.bfloat16
    row2 = edge_index[0].reshape(NCH, CH)
    col2 = edge_index[1].reshape(NCH, CH)
    zpad = jnp.zeros((N, PD - 3), jnp.float32)
    posp = jnp.concatenate([pos, zpad], axis=1)
    velp = jnp.concatenate([vel, zpad], axis=1)
    zz = jnp.zeros((STRIPE, F), jnp.float32)

    ldj = jnp.zeros((), jnp.float32)
    for i in range(2):
        W1a = W_e1[i, :D].astype(bf)
        W1b = W_e1[i, D:2 * D].astype(bf)
        w1r = W_e1[i, 2 * D:]
        b1 = b_e1[i].reshape(1, D)
        W2, b2 = W_e2[i].astype(bf), b_e2[i].reshape(1, D)
        Wc1, bc1, Wc2 = W_c1[i].astype(bf), b_c1[i].reshape(1, D), W_c2[i]
        Wv1, bv1 = W_v1[i], b_v1[i].reshape(1, D)
        Wv2, bv2 = W_v2[i], b_v2[i].reshape(1, 1)
        Wn1a, Wn1b = W_n1[i, :D], W_n1[i, D:]
        bn1, Wn2, bn2 = b_n1[i].reshape(1, D), W_n2[i], b_n2[i].reshape(1, D)

        av, bv_ = _tc_prep(h, W1a, W1b, b1)
        sv, pd = _sc_gather(av, bv_, posp, row2, col2)
        ef = _tc_edge(sv, pd, w1r, W2, b2, Wc1, bc1, Wc2)
        parts = _sc_scatter(ef, row2, zz)
        h, g, velp, posp, lds = _tc_node(h, g, velp, posp, parts,
                                         Wv1, bv1, Wv2, bv2,
                                         Wn1a, Wn1b, bn1, Wn2, bn2)
        ldj = ldj + lds[0, 0]

    return (h, g, posp[:, :3], velp[:, :3], ldj)


# speedup vs baseline: 2.3843x; 1.3633x over previous
import functools

import jax
import jax.numpy as jnp
from jax import lax
from jax.experimental import pallas as pl
from jax.experimental.pallas import tpu as pltpu
from jax.experimental.pallas import tpu_sc as plsc

N = 10000
E = 160000
D = 128
PD = 16
F = D + PD
CNT_LANE = 8
CH = 128
NCH = E // CH
NC = 2
NS = 16
NW = NC * NS
STRIPE = N // NS
DT = 0.001
DH = 0.001

BE = 2000
BN = 2000

_mesh = plsc.VectorSubcoreMesh(core_axis_name="c", subcore_axis_name="s")


@functools.partial(
    pl.kernel,
    out_type=(
        jax.ShapeDtypeStruct((E, D), jnp.float32),
        jax.ShapeDtypeStruct((E, D), jnp.float32),
        jax.ShapeDtypeStruct((E, PD), jnp.float32),
    ),
    mesh=_mesh,
    scratch_types=[
        pltpu.VMEM((CH,), jnp.int32),
        pltpu.VMEM((CH,), jnp.int32),
        pltpu.VMEM((CH, D), jnp.float32),
        pltpu.VMEM((CH, D), jnp.float32),
        pltpu.VMEM((CH, PD), jnp.float32),
        pltpu.VMEM((CH, PD), jnp.float32),
        pltpu.SemaphoreType.DMA,
        pltpu.SemaphoreType.DMA,
        pltpu.SemaphoreType.DMA,
        pltpu.SemaphoreType.DMA,
    ],
    compiler_params=pltpu.CompilerParams(use_tc_tiling_on_sc=False),
)
def _sc_gather(a_hbm, b_hbm, posp_hbm, row_hbm, col_hbm, ar_hbm, bc_hbm,
               pd_hbm, idxr, idxc, av, bv, prv, pcv, s0, s1, s2, s3):
    wid = lax.axis_index("s") * NC + lax.axis_index("c")
    nck = (NCH - wid + NW - 1) // NW

    def chunk(k, carry):
        j = wid + k * NW
        base = j * CH
        pltpu.sync_copy(row_hbm.at[j], idxr)
        pltpu.sync_copy(col_hbm.at[j], idxc)
        cp0 = pltpu.async_copy(a_hbm.at[idxr], av, s0)
        cp1 = pltpu.async_copy(b_hbm.at[idxc], bv, s1)
        cp2 = pltpu.async_copy(posp_hbm.at[idxr], prv, s2)
        cp3 = pltpu.async_copy(posp_hbm.at[idxc], pcv, s3)
        cp2.wait()
        cp3.wait()

        def drow(i, c2):
            prv[i, :] = prv[i, :] - pcv[i, :]
            return c2

        lax.fori_loop(0, CH, drow, 0, unroll=4)
        cp0.wait()
        cp1.wait()
        pltpu.sync_copy(av, ar_hbm.at[pl.ds(base, CH)])
        pltpu.sync_copy(bv, bc_hbm.at[pl.ds(base, CH)])
        pltpu.sync_copy(prv, pd_hbm.at[pl.ds(base, CH)])
        return carry

    lax.fori_loop(0, nck, chunk, 0)


@functools.partial(
    pl.kernel,
    out_type=(
        jax.ShapeDtypeStruct((NC, N, D), jnp.float32),
        jax.ShapeDtypeStruct((NC, N, PD), jnp.float32),
    ),
    mesh=_mesh,
    scratch_types=[
        pltpu.VMEM((CH, D), jnp.float32),
        pltpu.VMEM((CH, PD), jnp.float32),
        pltpu.VMEM((CH,), jnp.int32),
        pltpu.VMEM_SHARED((N, D), jnp.float32),
        pltpu.VMEM_SHARED((N, PD), jnp.float32),
    ],
    compiler_params=pltpu.CompilerParams(use_tc_tiling_on_sc=False),
)
def _sc_scatter(e_hbm, tr_hbm, row_hbm, z_hbm, zt_hbm, parts_hbm, parts2_hbm,
                ev, trv, idxv, acc, acc2):
    cid = lax.axis_index("c")
    sid = lax.axis_index("s")
    wid = sid * NC + cid
    pltpu.sync_copy(z_hbm, acc.at[pl.ds(sid * STRIPE, STRIPE)])
    pltpu.sync_copy(zt_hbm, acc2.at[pl.ds(sid * STRIPE, STRIPE)])
    plsc.subcore_barrier()

    nck = (NCH - wid + NW - 1) // NW

    def chunk(k, carry):
        j = wid + k * NW
        pltpu.sync_copy(row_hbm.at[j], idxv)
        pltpu.sync_copy(e_hbm.at[pl.ds(j * CH, CH)], ev)
        pltpu.sync_copy(tr_hbm.at[pl.ds(j * CH, CH)], trv)
        pltpu.sync_copy(ev, acc.at[idxv], add=True)
        pltpu.sync_copy(trv, acc2.at[idxv], add=True)
        return carry

    lax.fori_loop(0, nck, chunk, 0)
    plsc.subcore_barrier()
    pltpu.sync_copy(acc.at[pl.ds(sid * STRIPE, STRIPE)],
                    parts_hbm.at[cid, pl.ds(sid * STRIPE, STRIPE)])
    pltpu.sync_copy(acc2.at[pl.ds(sid * STRIPE, STRIPE)],
                    parts2_hbm.at[cid, pl.ds(sid * STRIPE, STRIPE)])


def _prep_body(h, W1a, W1b, b1, a_out, b_out):
    hb = h[...].astype(jnp.bfloat16)
    a_out[...] = (jnp.dot(hb, W1a[...], preferred_element_type=jnp.float32)
                  + b1[...])
    b_out[...] = jnp.dot(hb, W1b[...], preferred_element_type=jnp.float32)


def _tc_prep(h, W1a, W1b, b1):
    nb = N // BN
    wspec = lambda shape: pl.BlockSpec(shape, lambda i: (0, 0))
    return pl.pallas_call(
        _prep_body,
        grid=(nb,),
        in_specs=[
            pl.BlockSpec((BN, D), lambda i: (i, 0)),
            wspec((D, D)), wspec((D, D)), wspec((1, D)),
        ],
        out_specs=[
            pl.BlockSpec((BN, D), lambda i: (i, 0)),
            pl.BlockSpec((BN, D), lambda i: (i, 0)),
        ],
        out_shape=[
            jax.ShapeDtypeStruct((N, D), jnp.float32),
            jax.ShapeDtypeStruct((N, D), jnp.float32),
        ],
        compiler_params=pltpu.CompilerParams(
            dimension_semantics=("arbitrary",)),
    )(h, W1a, W1b, b1)


def _edge_body(ar, bc, pd, w1r, W2, b2, Wc1, bc1, Wc2, out_e, out_tr):
    f32 = jnp.float32
    bf = jnp.bfloat16
    pdv = pd[...]
    radial = jnp.sum(pdv * pdv, axis=1, keepdims=True)
    x = ar[...] + bc[...] + radial * w1r[...]
    x = x * jax.nn.sigmoid(x)
    x = jnp.dot(x.astype(bf), W2[...], preferred_element_type=f32) + b2[...]
    e = x * jax.nn.sigmoid(x)
    eb = e.astype(bf)
    y = jnp.dot(eb, Wc1[...], preferred_element_type=f32) + bc1[...]
    y = y * jax.nn.sigmoid(y)
    cw = y @ Wc2[...]
    tr = jnp.clip(pdv * cw, -100.0, 100.0)
    lane = lax.broadcasted_iota(jnp.int32, (1, PD), 1)
    tr = jnp.where(lane == CNT_LANE, 1.0, tr)
    out_e[...] = e
    out_tr[...] = tr


def _tc_edge(ar, bc, pd, w1r, W2, b2, Wc1, bc1, Wc2):
    nb = E // BE
    wspec = lambda shape: pl.BlockSpec(shape, lambda i: (0, 0))
    return pl.pallas_call(
        _edge_body,
        grid=(nb,),
        in_specs=[
            pl.BlockSpec((BE, D), lambda i: (i, 0)),
            pl.BlockSpec((BE, D), lambda i: (i, 0)),
            pl.BlockSpec((BE, PD), lambda i: (i, 0)),
            wspec((1, D)),
            wspec((D, D)), wspec((1, D)),
            wspec((D, D)), wspec((1, D)), wspec((D, 1)),
        ],
        out_specs=[
            pl.BlockSpec((BE, D), lambda i: (i, 0)),
            pl.BlockSpec((BE, PD), lambda i: (i, 0)),
        ],
        out_shape=[
            jax.ShapeDtypeStruct((E, D), jnp.float32),
            jax.ShapeDtypeStruct((E, PD), jnp.float32),
        ],
        compiler_params=pltpu.CompilerParams(
            dimension_semantics=("arbitrary",)),
    )(ar, bc, pd, w1r, W2, b2, Wc1, bc1, Wc2)


def _node_body(h, g, velp, posp, parts, parts2, Wv1, bv1, Wv2, bv2,
               Wn1a, Wn1b, bn1, Wn2, bn2,
               h2, g2, velp2, posp2, ldj):
    agg = parts[0] + parts[1]
    st = parts2[0] + parts2[1]
    lane = lax.broadcasted_iota(jnp.int32, (1, PD), 1)
    cnt = jnp.sum(jnp.where(lane == CNT_LANE, st, 0.0), axis=1, keepdims=True)
    force = st / jnp.clip(cnt, 1.0, None)
    force = jnp.where(lane < 3, force, 0.0)

    hv = h[...]
    sv = hv @ Wv1[...] + bv1[...]
    sv = sv * jax.nn.sigmoid(sv)
    sv = sv @ Wv2[...] + bv2[...]

    x = hv @ Wn1a[...] + agg @ Wn1b[...] + bn1[...]
    x = x * jax.nn.sigmoid(x)
    no = x @ Wn2[...] + bn2[...]

    vel_new = jnp.exp(sv) * velp[...] + force * DT
    posp2[...] = posp[...] + vel_new * DT
    velp2[...] = vel_new
    g_new = g[...] + no * DH
    g2[...] = g_new
    h2[...] = hv + g_new * DH

    @pl.when(pl.program_id(0) == 0)
    def _():
        ldj[...] = jnp.zeros_like(ldj)

    ldj[...] += jnp.sum(sv)


def _tc_node(h, g, velp, posp, parts, parts2, Wv1, bv1, Wv2, bv2,
             Wn1a, Wn1b, bn1, Wn2, bn2):
    nb = N // BN
    wspec = lambda shape: pl.BlockSpec(shape, lambda i: tuple(0 for _ in shape))
    return pl.pallas_call(
        _node_body,
        grid=(nb,),
        in_specs=[
            pl.BlockSpec((BN, D), lambda i: (i, 0)),
            pl.BlockSpec((BN, D), lambda i: (i, 0)),
            pl.BlockSpec((BN, PD), lambda i: (i, 0)),
            pl.BlockSpec((BN, PD), lambda i: (i, 0)),
            pl.BlockSpec((NC, BN, D), lambda i: (0, i, 0)),
            pl.BlockSpec((NC, BN, PD), lambda i: (0, i, 0)),
            wspec((D, D)), wspec((1, D)), wspec((D, 1)), wspec((1, 1)),
            wspec((D, D)), wspec((D, D)), wspec((1, D)),
            wspec((D, D)), wspec((1, D)),
        ],
        out_specs=[
            pl.BlockSpec((BN, D), lambda i: (i, 0)),
            pl.BlockSpec((BN, D), lambda i: (i, 0)),
            pl.BlockSpec((BN, PD), lambda i: (i, 0)),
            pl.BlockSpec((BN, PD), lambda i: (i, 0)),
            pl.BlockSpec((1, 1), lambda i: (0, 0)),
        ],
        out_shape=[
            jax.ShapeDtypeStruct((N, D), jnp.float32),
            jax.ShapeDtypeStruct((N, D), jnp.float32),
            jax.ShapeDtypeStruct((N, PD), jnp.float32),
            jax.ShapeDtypeStruct((N, PD), jnp.float32),
            jax.ShapeDtypeStruct((1, 1), jnp.float32),
        ],
        compiler_params=pltpu.CompilerParams(
            dimension_semantics=("arbitrary",)),
    )(h, g, velp, posp, parts, parts2, Wv1, bv1, Wv2, bv2,
      Wn1a, Wn1b, bn1, Wn2, bn2)


def kernel(h, pos, g, vel, edge_index, W_e1, b_e1, W_e2, b_e2, W_n1, b_n1,
           W_n2, b_n2, W_c1, b_c1, W_c2, W_v1, b_v1, W_v2, b_v2):
    bf = jnp.bfloat16
    row2 = edge_index[0].reshape(NCH, CH)
    col2 = edge_index[1].reshape(NCH, CH)
    zpad = jnp.zeros((N, PD - 3), jnp.float32)
    posp = jnp.concatenate([pos, zpad], axis=1)
    velp = jnp.concatenate([vel, zpad], axis=1)
    zz = jnp.zeros((STRIPE, D), jnp.float32)
    zzt = jnp.zeros((STRIPE, PD), jnp.float32)

    ldj = jnp.zeros((), jnp.float32)
    for i in range(2):
        W1a = W_e1[i, :D].astype(bf)
        W1b = W_e1[i, D:2 * D].astype(bf)
        w1r = W_e1[i, 2 * D:]
        b1 = b_e1[i].reshape(1, D)
        W2, b2 = W_e2[i].astype(bf), b_e2[i].reshape(1, D)
        Wc1, bc1, Wc2 = W_c1[i].astype(bf), b_c1[i].reshape(1, D), W_c2[i]
        Wv1, bv1 = W_v1[i], b_v1[i].reshape(1, D)
        Wv2, bv2 = W_v2[i], b_v2[i].reshape(1, 1)
        Wn1a, Wn1b = W_n1[i, :D], W_n1[i, D:]
        bn1, Wn2, bn2 = b_n1[i].reshape(1, D), W_n2[i], b_n2[i].reshape(1, D)

        av, bv_ = _tc_prep(h, W1a, W1b, b1)
        ar, bc, pd = _sc_gather(av, bv_, posp, row2, col2)
        ev, trv = _tc_edge(ar, bc, pd, w1r, W2, b2, Wc1, bc1, Wc2)
        parts, parts2 = _sc_scatter(ev, trv, row2, zz, zzt)
        h, g, velp, posp, lds = _tc_node(h, g, velp, posp, parts, parts2,
                                         Wv1, bv1, Wv2, bv2,
                                         Wn1a, Wn1b, bn1, Wn2, bn2)
        ldj = ldj + lds[0, 0]

    return (h, g, posp[:, :3], velp[:, :3], ldj)


# speedup vs baseline: 2.7441x; 1.1509x over previous
import functools

import jax
import jax.numpy as jnp
from jax import lax
from jax.experimental import pallas as pl
from jax.experimental.pallas import tpu as pltpu
from jax.experimental.pallas import tpu_sc as plsc

N = 10000
E = 160000
D = 128
PD = 16
F = D + PD
CNT_LANE = 8
CH = 128
NCH = E // CH
NC = 2
NS = 16
NW = NC * NS
STRIPE = N // NS
DT = 0.001
DH = 0.001

BE = 2000
BN = 2000

_mesh = plsc.VectorSubcoreMesh(core_axis_name="c", subcore_axis_name="s")


def _make_gather(nch):
    ne = nch * CH

    @functools.partial(
        pl.kernel,
        out_type=(
            jax.ShapeDtypeStruct((ne, D), jnp.float32),
            jax.ShapeDtypeStruct((ne, D), jnp.float32),
            jax.ShapeDtypeStruct((ne, PD), jnp.float32),
        ),
        mesh=_mesh,
        scratch_types=[
            pltpu.VMEM((CH,), jnp.int32),
            pltpu.VMEM((CH,), jnp.int32),
            pltpu.VMEM((CH, D), jnp.float32),
            pltpu.VMEM((CH, D), jnp.float32),
            pltpu.VMEM((CH, PD), jnp.float32),
            pltpu.VMEM((CH, PD), jnp.float32),
            pltpu.SemaphoreType.DMA,
            pltpu.SemaphoreType.DMA,
            pltpu.SemaphoreType.DMA,
            pltpu.SemaphoreType.DMA,
        ],
        compiler_params=pltpu.CompilerParams(use_tc_tiling_on_sc=False),
    )
    def _g(a_hbm, b_hbm, posp_hbm, row_hbm, col_hbm, ar_hbm, bc_hbm,
           pd_hbm, idxr, idxc, av, bv, prv, pcv, s0, s1, s2, s3):
        wid = lax.axis_index("s") * NC + lax.axis_index("c")
        nck = (nch - wid + NW - 1) // NW

        def chunk(k, carry):
            j = wid + k * NW
            base = j * CH
            pltpu.sync_copy(row_hbm.at[j], idxr)
            pltpu.sync_copy(col_hbm.at[j], idxc)
            cp0 = pltpu.async_copy(a_hbm.at[idxr], av, s0)
            cp1 = pltpu.async_copy(b_hbm.at[idxc], bv, s1)
            cp2 = pltpu.async_copy(posp_hbm.at[idxr], prv, s2)
            cp3 = pltpu.async_copy(posp_hbm.at[idxc], pcv, s3)
            cp2.wait()
            cp3.wait()

            def drow(i, c2):
                prv[i, :] = prv[i, :] - pcv[i, :]
                return c2

            lax.fori_loop(0, CH, drow, 0, unroll=4)
            cp0.wait()
            cp1.wait()
            pltpu.sync_copy(av, ar_hbm.at[pl.ds(base, CH)])
            pltpu.sync_copy(bv, bc_hbm.at[pl.ds(base, CH)])
            pltpu.sync_copy(prv, pd_hbm.at[pl.ds(base, CH)])
            return carry

        lax.fori_loop(0, nck, chunk, 0)

    return _g


_gather_half = _make_gather(NCH // 2)


def _make_scatter(nch):
    @functools.partial(
        pl.kernel,
        out_type=(
            jax.ShapeDtypeStruct((NC, N, D), jnp.float32),
            jax.ShapeDtypeStruct((NC, N, PD), jnp.float32),
        ),
        mesh=_mesh,
        scratch_types=[
            pltpu.VMEM((CH, D), jnp.float32),
            pltpu.VMEM((CH, PD), jnp.float32),
            pltpu.VMEM((CH,), jnp.int32),
            pltpu.VMEM_SHARED((N, D), jnp.float32),
            pltpu.VMEM_SHARED((N, PD), jnp.float32),
        ],
        compiler_params=pltpu.CompilerParams(use_tc_tiling_on_sc=False),
    )
    def _s(e_hbm, tr_hbm, row_hbm, z_hbm, zt_hbm, parts_hbm, parts2_hbm,
           ev, trv, idxv, acc, acc2):
        cid = lax.axis_index("c")
        sid = lax.axis_index("s")
        wid = sid * NC + cid
        pltpu.sync_copy(z_hbm, acc.at[pl.ds(sid * STRIPE, STRIPE)])
        pltpu.sync_copy(zt_hbm, acc2.at[pl.ds(sid * STRIPE, STRIPE)])
        plsc.subcore_barrier()

        nck = (nch - wid + NW - 1) // NW

        def chunk(k, carry):
            j = wid + k * NW
            pltpu.sync_copy(row_hbm.at[j], idxv)
            pltpu.sync_copy(e_hbm.at[pl.ds(j * CH, CH)], ev)
            pltpu.sync_copy(tr_hbm.at[pl.ds(j * CH, CH)], trv)
            pltpu.sync_copy(ev, acc.at[idxv], add=True)
            pltpu.sync_copy(trv, acc2.at[idxv], add=True)
            return carry

        lax.fori_loop(0, nck, chunk, 0)
        plsc.subcore_barrier()
        pltpu.sync_copy(acc.at[pl.ds(sid * STRIPE, STRIPE)],
                        parts_hbm.at[cid, pl.ds(sid * STRIPE, STRIPE)])
        pltpu.sync_copy(acc2.at[pl.ds(sid * STRIPE, STRIPE)],
                        parts2_hbm.at[cid, pl.ds(sid * STRIPE, STRIPE)])

    return _s


_scatter_half = _make_scatter(NCH // 2)


def _prep_body(h, W1a, W1b, b1, a_out, b_out):
    hb = h[...].astype(jnp.bfloat16)
    a_out[...] = (jnp.dot(hb, W1a[...], preferred_element_type=jnp.float32)
                  + b1[...])
    b_out[...] = jnp.dot(hb, W1b[...], preferred_element_type=jnp.float32)


def _tc_prep(h, W1a, W1b, b1):
    nb = N // BN
    wspec = lambda shape: pl.BlockSpec(shape, lambda i: (0, 0))
    return pl.pallas_call(
        _prep_body,
        grid=(nb,),
        in_specs=[
            pl.BlockSpec((BN, D), lambda i: (i, 0)),
            wspec((D, D)), wspec((D, D)), wspec((1, D)),
        ],
        out_specs=[
            pl.BlockSpec((BN, D), lambda i: (i, 0)),
            pl.BlockSpec((BN, D), lambda i: (i, 0)),
        ],
        out_shape=[
            jax.ShapeDtypeStruct((N, D), jnp.float32),
            jax.ShapeDtypeStruct((N, D), jnp.float32),
        ],
        compiler_params=pltpu.CompilerParams(
            dimension_semantics=("arbitrary",)),
    )(h, W1a, W1b, b1)


def _edge_body(ar, bc, pd, w1r, W2, b2, Wc1, bc1, Wc2, out_e, out_tr):
    f32 = jnp.float32
    bf = jnp.bfloat16
    pdv = pd[...]
    radial = jnp.sum(pdv * pdv, axis=1, keepdims=True)
    x = ar[...] + bc[...] + radial * w1r[...]
    x = x * jax.nn.sigmoid(x)
    x = jnp.dot(x.astype(bf), W2[...], preferred_element_type=f32) + b2[...]
    e = x * jax.nn.sigmoid(x)
    eb = e.astype(bf)
    y = jnp.dot(eb, Wc1[...], preferred_element_type=f32) + bc1[...]
    y = y * jax.nn.sigmoid(y)
    cw = y @ Wc2[...]
    tr = jnp.clip(pdv * cw, -100.0, 100.0)
    lane = lax.broadcasted_iota(jnp.int32, (1, PD), 1)
    tr = jnp.where(lane == CNT_LANE, 1.0, tr)
    out_e[...] = e
    out_tr[...] = tr


def _tc_edge(ar, bc, pd, w1r, W2, b2, Wc1, bc1, Wc2):
    ne = ar.shape[0]
    nb = ne // BE
    wspec = lambda shape: pl.BlockSpec(shape, lambda i: (0, 0))
    return pl.pallas_call(
        _edge_body,
        grid=(nb,),
        in_specs=[
            pl.BlockSpec((BE, D), lambda i: (i, 0)),
            pl.BlockSpec((BE, D), lambda i: (i, 0)),
            pl.BlockSpec((BE, PD), lambda i: (i, 0)),
            wspec((1, D)),
            wspec((D, D)), wspec((1, D)),
            wspec((D, D)), wspec((1, D)), wspec((D, 1)),
        ],
        out_specs=[
            pl.BlockSpec((BE, D), lambda i: (i, 0)),
            pl.BlockSpec((BE, PD), lambda i: (i, 0)),
        ],
        out_shape=[
            jax.ShapeDtypeStruct((ne, D), jnp.float32),
            jax.ShapeDtypeStruct((ne, PD), jnp.float32),
        ],
        compiler_params=pltpu.CompilerParams(
            dimension_semantics=("arbitrary",)),
    )(ar, bc, pd, w1r, W2, b2, Wc1, bc1, Wc2)


def _node_body(h, g, velp, posp, parts, parts2, partsb, parts2b,
               Wv1, bv1, Wv2, bv2,
               Wn1a, Wn1b, bn1, Wn2, bn2,
               h2, g2, velp2, posp2, ldj):
    agg = (parts[0] + parts[1]) + (partsb[0] + partsb[1])
    st = (parts2[0] + parts2[1]) + (parts2b[0] + parts2b[1])
    lane = lax.broadcasted_iota(jnp.int32, (1, PD), 1)
    cnt = jnp.sum(jnp.where(lane == CNT_LANE, st, 0.0), axis=1, keepdims=True)
    force = st / jnp.clip(cnt, 1.0, None)
    force = jnp.where(lane < 3, force, 0.0)

    hv = h[...]
    sv = hv @ Wv1[...] + bv1[...]
    sv = sv * jax.nn.sigmoid(sv)
    sv = sv @ Wv2[...] + bv2[...]

    x = hv @ Wn1a[...] + agg @ Wn1b[...] + bn1[...]
    x = x * jax.nn.sigmoid(x)
    no = x @ Wn2[...] + bn2[...]

    vel_new = jnp.exp(sv) * velp[...] + force * DT
    posp2[...] = posp[...] + vel_new * DT
    velp2[...] = vel_new
    g_new = g[...] + no * DH
    g2[...] = g_new
    h2[...] = hv + g_new * DH

    @pl.when(pl.program_id(0) == 0)
    def _():
        ldj[...] = jnp.zeros_like(ldj)

    ldj[...] += jnp.sum(sv)


def _tc_node(h, g, velp, posp, parts, parts2, partsb, parts2b,
             Wv1, bv1, Wv2, bv2, Wn1a, Wn1b, bn1, Wn2, bn2):
    nb = N // BN
    wspec = lambda shape: pl.BlockSpec(shape, lambda i: tuple(0 for _ in shape))
    return pl.pallas_call(
        _node_body,
        grid=(nb,),
        in_specs=[
            pl.BlockSpec((BN, D), lambda i: (i, 0)),
            pl.BlockSpec((BN, D), lambda i: (i, 0)),
            pl.BlockSpec((BN, PD), lambda i: (i, 0)),
            pl.BlockSpec((BN, PD), lambda i: (i, 0)),
            pl.BlockSpec((NC, BN, D), lambda i: (0, i, 0)),
            pl.BlockSpec((NC, BN, PD), lambda i: (0, i, 0)),
            pl.BlockSpec((NC, BN, D), lambda i: (0, i, 0)),
            pl.BlockSpec((NC, BN, PD), lambda i: (0, i, 0)),
            wspec((D, D)), wspec((1, D)), wspec((D, 1)), wspec((1, 1)),
            wspec((D, D)), wspec((D, D)), wspec((1, D)),
            wspec((D, D)), wspec((1, D)),
        ],
        out_specs=[
            pl.BlockSpec((BN, D), lambda i: (i, 0)),
            pl.BlockSpec((BN, D), lambda i: (i, 0)),
            pl.BlockSpec((BN, PD), lambda i: (i, 0)),
            pl.BlockSpec((BN, PD), lambda i: (i, 0)),
            pl.BlockSpec((1, 1), lambda i: (0, 0)),
        ],
        out_shape=[
            jax.ShapeDtypeStruct((N, D), jnp.float32),
            jax.ShapeDtypeStruct((N, D), jnp.float32),
            jax.ShapeDtypeStruct((N, PD), jnp.float32),
            jax.ShapeDtypeStruct((N, PD), jnp.float32),
            jax.ShapeDtypeStruct((1, 1), jnp.float32),
        ],
        compiler_params=pltpu.CompilerParams(
            dimension_semantics=("arbitrary",)),
    )(h, g, velp, posp, parts, parts2, partsb, parts2b,
      Wv1, bv1, Wv2, bv2, Wn1a, Wn1b, bn1, Wn2, bn2)


def kernel(h, pos, g, vel, edge_index, W_e1, b_e1, W_e2, b_e2, W_n1, b_n1,
           W_n2, b_n2, W_c1, b_c1, W_c2, W_v1, b_v1, W_v2, b_v2):
    bf = jnp.bfloat16
    NH = NCH // 2
    row2 = edge_index[0].reshape(NCH, CH)
    col2 = edge_index[1].reshape(NCH, CH)
    rowh = (row2[:NH], row2[NH:])
    colh = (col2[:NH], col2[NH:])
    zpad = jnp.zeros((N, PD - 3), jnp.float32)
    posp = jnp.concatenate([pos, zpad], axis=1)
    velp = jnp.concatenate([vel, zpad], axis=1)
    zz = jnp.zeros((STRIPE, D), jnp.float32)
    zzt = jnp.zeros((STRIPE, PD), jnp.float32)

    ldj = jnp.zeros((), jnp.float32)
    for i in range(2):
        W1a = W_e1[i, :D].astype(bf)
        W1b = W_e1[i, D:2 * D].astype(bf)
        w1r = W_e1[i, 2 * D:]
        b1 = b_e1[i].reshape(1, D)
        W2, b2 = W_e2[i].astype(bf), b_e2[i].reshape(1, D)
        Wc1, bc1, Wc2 = W_c1[i].astype(bf), b_c1[i].reshape(1, D), W_c2[i]
        Wv1, bv1 = W_v1[i], b_v1[i].reshape(1, D)
        Wv2, bv2 = W_v2[i], b_v2[i].reshape(1, 1)
        Wn1a, Wn1b = W_n1[i, :D], W_n1[i, D:]
        bn1, Wn2, bn2 = b_n1[i].reshape(1, D), W_n2[i], b_n2[i].reshape(1, D)

        av, bv_ = _tc_prep(h, W1a, W1b, b1)
        g0 = _gather_half(av, bv_, posp, rowh[0], colh[0])
        g1 = _gather_half(av, bv_, posp, rowh[1], colh[1])
        e0 = _tc_edge(g0[0], g0[1], g0[2], w1r, W2, b2, Wc1, bc1, Wc2)
        e1 = _tc_edge(g1[0], g1[1], g1[2], w1r, W2, b2, Wc1, bc1, Wc2)
        p0, q0 = _scatter_half(e0[0], e0[1], rowh[0], zz, zzt)
        p1, q1 = _scatter_half(e1[0], e1[1], rowh[1], zz, zzt)
        h, g, velp, posp, lds = _tc_node(h, g, velp, posp, p0, q0, p1, q1,
                                         Wv1, bv1, Wv2, bv2,
                                         Wn1a, Wn1b, bn1, Wn2, bn2)
        ldj = ldj + lds[0, 0]

    return (h, g, posp[:, :3], velp[:, :3], ldj)
